# baseline (speedup 1.0000x reference)
import functools

import jax
import jax.numpy as jnp
from jax import lax
from jax.experimental import pallas as pl
from jax.experimental.pallas import tpu as pltpu
from jax.experimental.pallas import tpu_sc as plsc

K = 256
_NW = 32
_L = 16


def _sc_counts(labs_flat):
    n = labs_flat.shape[0]
    lpw = n // _NW

    mesh = plsc.VectorSubcoreMesh(core_axis_name="c", subcore_axis_name="s")

    @functools.partial(
        pl.kernel,
        out_type=jax.ShapeDtypeStruct((_NW, K), jnp.float32),
        mesh=mesh,
        scratch_types=[
            pltpu.VMEM((lpw,), jnp.int32),
            pltpu.VMEM((K,), jnp.float32),
        ],
        compiler_params=pltpu.CompilerParams(needs_layout_passes=False),
    )
    def counts_kernel(labs_hbm, out_hbm, lab_v, hist_v):
        wid = lax.axis_index("s") * 2 + lax.axis_index("c")
        pltpu.sync_copy(labs_hbm.at[pl.ds(wid * lpw, lpw)], lab_v)
        zeros = jnp.zeros((_L,), jnp.float32)
        for i in range(K // _L):
            hist_v[pl.ds(i * _L, _L)] = zeros
        ones = jnp.ones((_L,), jnp.float32)

        @plsc.parallel_loop(0, lpw // _L, 1, unroll=8)
        def _hist(g):
            idx = lab_v[pl.ds(g * _L, _L)]
            plsc.addupdate_scatter(hist_v, [idx], ones)

        pltpu.sync_copy(hist_v, out_hbm.at[wid])

    return counts_kernel(labs_flat)


def _pool_body(nj, x_ref, lab_ref, cp_ref, out_ref):
    j = pl.program_id(1)

    labs = lab_ref[0]
    kiota = lax.broadcasted_iota(jnp.int32, (K, labs.shape[-1]), 0)
    onehot = (labs == kiota).astype(jnp.bfloat16)
    xb = x_ref[0].astype(jnp.bfloat16)

    psum = lax.dot_general(
        onehot, xb, (((1,), (1,)), ((), ())),
        preferred_element_type=jnp.float32)

    @pl.when(j == 0)
    def _init():
        out_ref[0] = psum

    @pl.when(j > 0)
    def _acc():
        out_ref[0] += psum

    @pl.when(j == nj - 1)
    def _finish():
        ones_col = jnp.ones((cp_ref.shape[1], 1), jnp.float32)
        counts = lax.dot_general(
            cp_ref[0], ones_col, (((0,), (0,)), ((), ())),
            preferred_element_type=jnp.float32)
        out_ref[0] = out_ref[0] / jnp.maximum(counts, 1.0)


def kernel(x, label_maps):
    B, C, H, W = x.shape
    HW = H * W
    chunk = 12544 if HW % 12544 == 0 else HW
    nj = HW // chunk
    wpi = _NW // B

    x3 = x.reshape(B, C, HW)
    labs = label_maps.reshape(B * nj, 1, chunk)

    partials = _sc_counts(label_maps.reshape(-1))
    cp = partials.reshape(B, wpi, K)

    out = pl.pallas_call(
        functools.partial(_pool_body, nj),
        grid=(B, nj),
        in_specs=[
            pl.BlockSpec((1, C, chunk), lambda b, j: (b, 0, j)),
            pl.BlockSpec((1, 1, chunk), lambda b, j: (b * nj + j, 0, 0)),
            pl.BlockSpec((1, wpi, K), lambda b, j: (b, 0, 0)),
        ],
        out_specs=pl.BlockSpec((1, K, C), lambda b, j: (b, 0, 0)),
        out_shape=jax.ShapeDtypeStruct((B, K, C), jnp.float32),
        compiler_params=pltpu.CompilerParams(
            dimension_semantics=("parallel", "arbitrary")),
    )(x3, labs, cp)
    return out
